# baseline (device time: 14531 ns/iter reference)
import functools
import os

import jax
import jax.numpy as jnp
from jax import lax
from jax.experimental import pallas as pl
from jax.experimental.pallas import tpu as pltpu

N_DEV = 32
K_CHUNK = 512
W_DEPTH = 6

_VARIANT = os.environ.get("KVARIANT", "full")
_FP8_DOT = os.environ.get("KFP8DOT", "1") == "1"

_sem_signal = getattr(pl, "semaphore_signal", None) or pltpu.semaphore_signal
_sem_wait = getattr(pl, "semaphore_wait", None) or pltpu.semaphore_wait
_CompilerParams = getattr(pltpu, "CompilerParams", None) or pltpu.TPUCompilerParams


def kernel(x, w_mat, scale_x, scale_w):
    m_total, k_per = x.shape
    k_total, n = w_mat.shape
    m_per = m_total // N_DEV
    comm_dtype = jnp.float8_e5m2
    mxu_dtype = jnp.float8_e5m2 if _FP8_DOT else jnp.bfloat16
    n_chunks = k_total // K_CHUNK
    blocks_per_chunk = K_CHUNK // k_per

    def body(x_ref, w_hbm, sx_ref, sw_ref, out_ref,
             x8_ref, xg_ref, wbuf_ref, wc_ref,
             send_sems, recv_sems, w_sems):
        my = lax.axis_index("i")

        if _VARIANT != "local":
            barrier = pltpu.get_barrier_semaphore()
            for d in range(1, N_DEV):
                _sem_signal(barrier, inc=1, device_id=((my + d) % N_DEV,),
                            device_id_type=pl.DeviceIdType.MESH)

        def _w_dma(c):
            return pltpu.make_async_copy(
                w_hbm.at[pl.ds(c * K_CHUNK, K_CHUNK), :],
                wbuf_ref.at[c % W_DEPTH],
                w_sems.at[c % W_DEPTH],
            )

        w_dmas = {}
        for c in range(min(W_DEPTH, n_chunks)):
            w_dmas[c] = _w_dma(c)
            w_dmas[c].start()

        x8_ref[:, :] = x_ref[:, :].astype(comm_dtype)

        if _VARIANT == "local":
            xg_ref[:, pl.ds(my * k_per, k_per)] = x8_ref[pl.ds(my * m_per, m_per), :]
        else:
            pltpu.make_async_copy(
                x8_ref.at[pl.ds(my * m_per, m_per), :],
                xg_ref.at[:, pl.ds(my * k_per, k_per)],
                recv_sems.at[my],
            ).start()

        sends = []
        if _VARIANT != "local":
            _sem_wait(barrier, N_DEV - 1)
            for d in range(1, N_DEV):
                dst = (my + d) % N_DEV
                rdma = pltpu.make_async_remote_copy(
                    src_ref=x8_ref.at[pl.ds(dst * m_per, m_per), :],
                    dst_ref=xg_ref.at[:, pl.ds(my * k_per, k_per)],
                    send_sem=send_sems.at[d - 1],
                    recv_sem=recv_sems.at[my],
                    device_id=(dst,),
                    device_id_type=pl.DeviceIdType.MESH,
                )
                rdma.start()
                sends.append(rdma)

        s = sx_ref[0] * sw_ref[0]

        for c in range(n_chunks):
            w_dmas[c].wait()
            wc_ref[c % 2] = wbuf_ref[c % W_DEPTH].astype(mxu_dtype)
            nxt = c + W_DEPTH
            if nxt < n_chunks:
                w_dmas[nxt] = _w_dma(nxt)
                w_dmas[nxt].start()

            if _VARIANT != "local":
                for j in range(c * blocks_per_chunk, (c + 1) * blocks_per_chunk):
                    pltpu.make_async_copy(
                        x8_ref.at[pl.ds(0, m_per), :],
                        xg_ref.at[:, pl.ds(j * k_per, k_per)],
                        recv_sems.at[j],
                    ).wait()

            part = jnp.dot(
                xg_ref[:, pl.ds(c * K_CHUNK, K_CHUNK)].astype(mxu_dtype),
                wc_ref[c % 2],
                preferred_element_type=jnp.float32,
            )
            if c == 0:
                out_ref[:, :] = part
            else:
                out_ref[:, :] = out_ref[:, :] + part

        out_ref[:, :] = jnp.maximum(out_ref[:, :] * s, 0.0)

        for rdma in sends:
            rdma.wait_send()

    return pl.pallas_call(
        body,
        out_shape=jax.ShapeDtypeStruct((m_per, n), jnp.float32),
        in_specs=[
            pl.BlockSpec(memory_space=pltpu.MemorySpace.VMEM),
            pl.BlockSpec(memory_space=pl.ANY),
            pl.BlockSpec(memory_space=pltpu.MemorySpace.SMEM),
            pl.BlockSpec(memory_space=pltpu.MemorySpace.SMEM),
        ],
        out_specs=pl.BlockSpec(memory_space=pltpu.MemorySpace.VMEM),
        scratch_shapes=[
            pltpu.VMEM((m_total, k_per), comm_dtype),
            pltpu.VMEM((m_per, k_total), comm_dtype),
            pltpu.VMEM((W_DEPTH, K_CHUNK, n), jnp.float32),
            pltpu.VMEM((2, K_CHUNK, n), mxu_dtype),
            pltpu.SemaphoreType.DMA((N_DEV - 1,)),
            pltpu.SemaphoreType.DMA((N_DEV,)),
            pltpu.SemaphoreType.DMA((W_DEPTH,)),
        ],
        compiler_params=(_CompilerParams() if _VARIANT == "local"
                         else _CompilerParams(collective_id=0)),
    )(x, w_mat, scale_x, scale_w)
